# Initial kernel scaffold; baseline (speedup 1.0000x reference)
import jax
import jax.numpy as jnp
from jax.experimental import pallas as pl


def kernel(z1, z2_raw, eps, num_atoms, atomic_nums, batch, lscaled_lattice, W_mu, b_mu, W_sigma, b_sigma, W_latt, b_latt, W_atom, b_atom, W_num, b_num, W_p1, b_p1, gamma, beta, W_p2, b_p2, scaler_mean, scaler_std):
    raise NotImplementedError("write your pallas kernel here")


# speedup vs baseline: 47.5599x; 47.5599x over previous
import functools

import jax
import jax.numpy as jnp
from jax import lax
from jax.experimental import pallas as pl
from jax.experimental.pallas import tpu as pltpu
from jax.experimental.pallas import tpu_sc as plsc

B = 4096
D = 256
N_ATOM_CLASSES = 100
NUM_CLASSES = 41

_NC = 2
_NS = 16
_NW = _NC * _NS
_L = 16


def _tc_body(z1_ref, z2r_ref, eps_ref, nat_ref, latt_ref,
             wmu_ref, bmu_ref, wsig_ref, bsig_ref,
             wlatt_ref, blatt_ref, watom_ref, batom_ref,
             wnum_ref, bnum_ref, wp1_ref, bp1_ref,
             gamma_ref, beta_ref, wp2_ref, bp2_ref,
             smean_ref, sstd_ref,
             partial_ref, g_ref):
    f32 = jnp.float32
    z2r = z2r_ref[...]
    mu = jnp.dot(z2r, wmu_ref[...], preferred_element_type=f32) + bmu_ref[...]
    logvar = jnp.dot(z2r, wsig_ref[...], preferred_element_type=f32) + bsig_ref[...]
    z2 = eps_ref[...] * jnp.exp(0.5 * logvar) + mu

    kld = jnp.mean(
        -0.5 * jnp.sum(1.0 + logvar - mu * mu - jnp.exp(logvar),
                       axis=1, keepdims=True))

    h = jnp.dot(z1_ref[...], wp1_ref[...], preferred_element_type=f32) + bp1_ref[...]
    m = jnp.mean(h, axis=0, keepdims=True)
    v = jnp.mean((h - m) * (h - m), axis=0, keepdims=True)
    h = (h - m) / jnp.sqrt(v + 1e-5) * gamma_ref[...] + beta_ref[...]
    h = jnp.maximum(h, 0.0)
    p1 = jnp.dot(h, wp2_ref[...], preferred_element_type=f32) + bp2_ref[...]

    dot_pz = jnp.sum(p1 * z2, axis=1, keepdims=True)
    np1 = jnp.sqrt(jnp.sum(p1 * p1, axis=1, keepdims=True))
    nz2 = jnp.sqrt(jnp.sum(z2 * z2, axis=1, keepdims=True))
    den = jnp.maximum(np1 * nz2, 1e-8)
    cos_loss = -jnp.mean(dot_pz / den)

    pred_latt = jnp.dot(z2, wlatt_ref[...], preferred_element_type=f32) + blatt_ref[...]
    tgt = (latt_ref[...] - smean_ref[...]) / sstd_ref[...]
    dl = pred_latt - tgt
    latt_loss = jnp.mean(dl * dl) * 10.0

    ln = jnp.dot(z2, wnum_ref[...], preferred_element_type=f32) + bnum_ref[...]
    mx_n = jnp.max(ln, axis=1, keepdims=True)
    lse_n = mx_n + jnp.log(jnp.sum(jnp.exp(ln - mx_n), axis=1, keepdims=True))
    iota_n = lax.broadcasted_iota(jnp.int32, (B, NUM_CLASSES), 1)
    tl_n = jnp.sum(jnp.where(iota_n == nat_ref[...], ln, 0.0),
                   axis=1, keepdims=True)
    num_loss = jnp.mean(lse_n - tl_n)

    la = jnp.dot(z2, watom_ref[...], preferred_element_type=f32) + batom_ref[...]
    mx_a = jnp.max(la, axis=1, keepdims=True)
    lse_a = mx_a + jnp.log(jnp.sum(jnp.exp(la - mx_a), axis=1, keepdims=True))
    inv_n = 1.0 / nat_ref[...].astype(f32)
    g_ref[...] = la * (inv_n * (1.0 / B))

    total = cos_loss + latt_loss + kld + num_loss + jnp.mean(lse_a)
    partial_ref[...] = total.reshape(1, 1)


def _make_sc_kernel(n_atoms, c_per_w):
    mesh = plsc.VectorSubcoreMesh(core_axis_name="c", subcore_axis_name="s")

    @functools.partial(
        pl.kernel,
        out_type=jax.ShapeDtypeStruct((_NW, _L), jnp.float32),
        mesh=mesh,
        scratch_types=[
            pltpu.VMEM((c_per_w,), jnp.int32),
            pltpu.VMEM((c_per_w,), jnp.int32),
            pltpu.VMEM((c_per_w,), jnp.int32),
            pltpu.VMEM((c_per_w,), jnp.float32),
            pltpu.VMEM((_L,), jnp.float32),
            pltpu.SemaphoreType.DMA,
        ],
    )
    def sc_gather_sum(g_hbm, b_hbm, a_hbm, out_hbm, bv, av, fv, vv, accv, sem):
        wid = lax.axis_index("s") * _NC + lax.axis_index("c")
        base = wid * c_per_w
        pltpu.sync_copy(b_hbm.at[pl.ds(base, c_per_w)], bv)
        pltpu.sync_copy(a_hbm.at[pl.ds(base, c_per_w)], av)

        def fbody(j, carry):
            s = pl.ds(j * _L, _L)
            fv[s] = bv[s] * N_ATOM_CLASSES + av[s] - 1
            return carry

        lax.fori_loop(0, c_per_w // _L, fbody, 0)

        def gbody(j, carry):
            s = pl.ds(j * 128, 128)
            pltpu.async_copy(g_hbm.at[fv.at[s]], vv.at[s], sem).wait()
            return carry

        lax.fori_loop(0, c_per_w // 128, gbody, 0)

        nvalid = n_atoms - base

        def abody(j, acc):
            lane = lax.iota(jnp.int32, _L) + j * _L
            return acc + jnp.where(lane < nvalid, vv[pl.ds(j * _L, _L)], 0.0)

        acc = lax.fori_loop(0, c_per_w // _L, abody,
                            jnp.zeros((_L,), jnp.float32))
        accv[...] = acc
        pltpu.sync_copy(accv, out_hbm.at[wid])

    return sc_gather_sum


def kernel(z1, z2_raw, eps, num_atoms, atomic_nums, batch, lscaled_lattice,
           W_mu, b_mu, W_sigma, b_sigma, W_latt, b_latt, W_atom, b_atom,
           W_num, b_num, W_p1, b_p1, gamma, beta, W_p2, b_p2,
           scaler_mean, scaler_std):
    f32 = jnp.float32
    n_atoms = atomic_nums.shape[0]
    n_pad = -n_atoms % (_NW * 128)
    c_per_w = (n_atoms + n_pad) // _NW

    partial, g = pl.pallas_call(
        _tc_body,
        out_shape=[
            jax.ShapeDtypeStruct((1, 1), f32),
            jax.ShapeDtypeStruct((B, N_ATOM_CLASSES), f32),
        ],
    )(z1, z2_raw, eps,
      num_atoms.astype(jnp.int32).reshape(B, 1),
      lscaled_lattice,
      W_mu, b_mu.reshape(1, D), W_sigma, b_sigma.reshape(1, D),
      W_latt, b_latt.reshape(1, 6), W_atom, b_atom.reshape(1, N_ATOM_CLASSES),
      W_num, b_num.reshape(1, NUM_CLASSES), W_p1, b_p1.reshape(1, D),
      gamma.reshape(1, D), beta.reshape(1, D), W_p2, b_p2.reshape(1, D),
      scaler_mean.reshape(1, 6), scaler_std.reshape(1, 6))

    batch_p = jnp.concatenate(
        [batch, jnp.zeros((n_pad,), jnp.int32)])
    anum_p = jnp.concatenate(
        [atomic_nums, jnp.ones((n_pad,), jnp.int32)])

    sc_parts = _make_sc_kernel(n_atoms, c_per_w)(
        g.reshape(B * N_ATOM_CLASSES), batch_p, anum_p)

    return partial[0, 0] - jnp.sum(sc_parts)


# speedup vs baseline: 56.8667x; 1.1957x over previous
import functools

import jax
import jax.numpy as jnp
from jax import lax
from jax.experimental import pallas as pl
from jax.experimental.pallas import tpu as pltpu
from jax.experimental.pallas import tpu_sc as plsc

B = 4096
D = 256
N_ATOM_CLASSES = 100
NUM_CLASSES = 41

_NC = 2
_NS = 16
_NW = _NC * _NS
_L = 16


def _tc_body(z1_ref, z2r_ref, eps_ref, nat_ref, latt_ref,
             wmu_ref, bmu_ref, wsig_ref, bsig_ref,
             wlatt_ref, blatt_ref, watom_ref, batom_ref,
             wnum_ref, bnum_ref, wp1_ref, bp1_ref,
             gamma_ref, beta_ref, wp2_ref, bp2_ref,
             smean_ref, sstd_ref,
             partial_ref, g_ref):
    f32 = jnp.float32
    z2r = z2r_ref[...]
    mu = jnp.dot(z2r, wmu_ref[...], preferred_element_type=f32) + bmu_ref[...]
    logvar = jnp.dot(z2r, wsig_ref[...], preferred_element_type=f32) + bsig_ref[...]
    z2 = eps_ref[...] * jnp.exp(0.5 * logvar) + mu

    kld = jnp.mean(
        -0.5 * jnp.sum(1.0 + logvar - mu * mu - jnp.exp(logvar),
                       axis=1, keepdims=True))

    h = jnp.dot(z1_ref[...], wp1_ref[...], preferred_element_type=f32) + bp1_ref[...]
    m = jnp.mean(h, axis=0, keepdims=True)
    v = jnp.mean((h - m) * (h - m), axis=0, keepdims=True)
    h = (h - m) / jnp.sqrt(v + 1e-5) * gamma_ref[...] + beta_ref[...]
    h = jnp.maximum(h, 0.0)
    p1 = jnp.dot(h, wp2_ref[...], preferred_element_type=f32) + bp2_ref[...]

    dot_pz = jnp.sum(p1 * z2, axis=1, keepdims=True)
    np1 = jnp.sqrt(jnp.sum(p1 * p1, axis=1, keepdims=True))
    nz2 = jnp.sqrt(jnp.sum(z2 * z2, axis=1, keepdims=True))
    den = jnp.maximum(np1 * nz2, 1e-8)
    cos_loss = -jnp.mean(dot_pz / den)

    pred_latt = jnp.dot(z2, wlatt_ref[...], preferred_element_type=f32) + blatt_ref[...]
    tgt = (latt_ref[...] - smean_ref[...]) / sstd_ref[...]
    dl = pred_latt - tgt
    latt_loss = jnp.mean(dl * dl) * 10.0

    ln = jnp.dot(z2, wnum_ref[...], preferred_element_type=f32) + bnum_ref[...]
    mx_n = jnp.max(ln, axis=1, keepdims=True)
    lse_n = mx_n + jnp.log(jnp.sum(jnp.exp(ln - mx_n), axis=1, keepdims=True))
    iota_n = lax.broadcasted_iota(jnp.int32, (B, NUM_CLASSES), 1)
    tl_n = jnp.sum(jnp.where(iota_n == nat_ref[...], ln, 0.0),
                   axis=1, keepdims=True)
    num_loss = jnp.mean(lse_n - tl_n)

    la = jnp.dot(z2, watom_ref[...], preferred_element_type=f32) + batom_ref[...]
    mx_a = jnp.max(la, axis=1, keepdims=True)
    lse_a = mx_a + jnp.log(jnp.sum(jnp.exp(la - mx_a), axis=1, keepdims=True))
    inv_n = 1.0 / nat_ref[...].astype(f32)
    g_ref[...] = la * (inv_n * (1.0 / B))

    total = cos_loss + latt_loss + kld + num_loss + jnp.mean(lse_a)
    partial_ref[...] = total.reshape(1, 1)


def _make_sc_kernel(n_atoms, c_per_w):
    mesh = plsc.VectorSubcoreMesh(core_axis_name="c", subcore_axis_name="s")

    @functools.partial(
        pl.kernel,
        out_type=jax.ShapeDtypeStruct((_NW, _L), jnp.float32),
        mesh=mesh,
        scratch_types=[
            pltpu.VMEM((c_per_w,), jnp.int32),
            pltpu.VMEM((c_per_w,), jnp.int32),
            pltpu.VMEM((c_per_w,), jnp.int32),
            pltpu.VMEM((c_per_w,), jnp.float32),
            pltpu.VMEM((_L,), jnp.float32),
            pltpu.SemaphoreType.DMA,
            pltpu.SemaphoreType.DMA,
        ],
    )
    def sc_gather_sum(g_hbm, b_hbm, a_hbm, out_hbm, bv, av, fv, vv, accv,
                      sem_in, sem_g):
        wid = lax.axis_index("s") * _NC + lax.axis_index("c")
        base = wid * c_per_w
        cb = pltpu.async_copy(b_hbm.at[pl.ds(base, c_per_w)], bv, sem_in)
        ca = pltpu.async_copy(a_hbm.at[pl.ds(base, c_per_w)], av, sem_in)
        cb.wait()
        ca.wait()

        def fire(j, carry):
            for k in range(128 // _L):
                s = pl.ds(j * 128 + k * _L, _L)
                fv[s] = bv[s] * N_ATOM_CLASSES + av[s] - 1
            s128 = pl.ds(j * 128, 128)
            pltpu.async_copy(g_hbm.at[fv.at[s128]], vv.at[s128], sem_g)
            return carry

        lax.fori_loop(0, c_per_w // 128, fire, 0)

        pltpu.make_async_copy(g_hbm.at[pl.ds(0, c_per_w)], vv, sem_g).wait()

        nvalid = n_atoms - base

        def abody(j, acc):
            for k in range(128 // _L):
                off = j * 128 + k * _L
                lane = lax.iota(jnp.int32, _L) + off
                acc = acc + jnp.where(lane < nvalid, vv[pl.ds(off, _L)], 0.0)
            return acc

        acc = lax.fori_loop(0, c_per_w // 128, abody,
                            jnp.zeros((_L,), jnp.float32))
        accv[...] = acc
        pltpu.sync_copy(accv, out_hbm.at[wid])

    return sc_gather_sum


def kernel(z1, z2_raw, eps, num_atoms, atomic_nums, batch, lscaled_lattice,
           W_mu, b_mu, W_sigma, b_sigma, W_latt, b_latt, W_atom, b_atom,
           W_num, b_num, W_p1, b_p1, gamma, beta, W_p2, b_p2,
           scaler_mean, scaler_std):
    f32 = jnp.float32
    n_atoms = atomic_nums.shape[0]
    n_pad = -n_atoms % (_NW * 128)
    c_per_w = (n_atoms + n_pad) // _NW

    partial, g = pl.pallas_call(
        _tc_body,
        out_shape=[
            jax.ShapeDtypeStruct((1, 1), f32),
            jax.ShapeDtypeStruct((B, N_ATOM_CLASSES), f32),
        ],
    )(z1, z2_raw, eps,
      num_atoms.astype(jnp.int32).reshape(B, 1),
      lscaled_lattice,
      W_mu, b_mu.reshape(1, D), W_sigma, b_sigma.reshape(1, D),
      W_latt, b_latt.reshape(1, 6), W_atom, b_atom.reshape(1, N_ATOM_CLASSES),
      W_num, b_num.reshape(1, NUM_CLASSES), W_p1, b_p1.reshape(1, D),
      gamma.reshape(1, D), beta.reshape(1, D), W_p2, b_p2.reshape(1, D),
      scaler_mean.reshape(1, 6), scaler_std.reshape(1, 6))

    batch_p = jnp.concatenate(
        [batch, jnp.zeros((n_pad,), jnp.int32)])
    anum_p = jnp.concatenate(
        [atomic_nums, jnp.ones((n_pad,), jnp.int32)])

    sc_parts = _make_sc_kernel(n_atoms, c_per_w)(
        g.reshape(B * N_ATOM_CLASSES), batch_p, anum_p)

    return partial[0, 0] - jnp.sum(sc_parts)


# speedup vs baseline: 60.1285x; 1.0574x over previous
import functools

import jax
import jax.numpy as jnp
from jax import lax
from jax.experimental import pallas as pl
from jax.experimental.pallas import tpu as pltpu
from jax.experimental.pallas import tpu_sc as plsc

B = 4096
D = 256
N_ATOM_CLASSES = 100
NUM_CLASSES = 41
GL = 128

_NC = 2
_NS = 16
_NW = _NC * _NS
_L = 16


def _tc_body(z1_ref, z2r_ref, eps_ref, nat_ref, latt_ref,
             wmu_ref, wsig_ref, wlatt_ref, watom_ref, wnum_ref,
             wp1_ref, wp2_ref, par_ref,
             partial_ref, g_ref):
    f32 = jnp.float32
    b_mu = par_ref[0:1, :]
    b_sigma = par_ref[1:2, :]
    b_p1 = par_ref[2:3, :]
    gamma = par_ref[3:4, :]
    beta = par_ref[4:5, :]
    b_p2 = par_ref[5:6, :]
    b_latt = par_ref[6:7, 0:6]
    b_atom = par_ref[7:8, 0:N_ATOM_CLASSES]
    b_num = par_ref[8:9, 0:NUM_CLASSES]
    smean = par_ref[9:10, 0:6]
    sstd = par_ref[10:11, 0:6]

    z2r = z2r_ref[...]
    mu = jnp.dot(z2r, wmu_ref[...], preferred_element_type=f32) + b_mu
    logvar = jnp.dot(z2r, wsig_ref[...], preferred_element_type=f32) + b_sigma
    z2 = eps_ref[...] * jnp.exp(0.5 * logvar) + mu

    kld = jnp.mean(
        -0.5 * jnp.sum(1.0 + logvar - mu * mu - jnp.exp(logvar),
                       axis=1, keepdims=True))

    h = jnp.dot(z1_ref[...], wp1_ref[...], preferred_element_type=f32) + b_p1
    m = jnp.mean(h, axis=0, keepdims=True)
    v = jnp.mean((h - m) * (h - m), axis=0, keepdims=True)
    h = (h - m) / jnp.sqrt(v + 1e-5) * gamma + beta
    h = jnp.maximum(h, 0.0)
    p1 = jnp.dot(h, wp2_ref[...], preferred_element_type=f32) + b_p2

    dot_pz = jnp.sum(p1 * z2, axis=1, keepdims=True)
    np1 = jnp.sqrt(jnp.sum(p1 * p1, axis=1, keepdims=True))
    nz2 = jnp.sqrt(jnp.sum(z2 * z2, axis=1, keepdims=True))
    den = jnp.maximum(np1 * nz2, 1e-8)
    cos_loss = -jnp.mean(dot_pz / den)

    pred_latt = jnp.dot(z2, wlatt_ref[...], preferred_element_type=f32) + b_latt
    tgt = (latt_ref[...] - smean) / sstd
    dl = pred_latt - tgt
    latt_loss = jnp.mean(dl * dl) * 10.0

    ln = jnp.dot(z2, wnum_ref[...], preferred_element_type=f32) + b_num
    mx_n = jnp.max(ln, axis=1, keepdims=True)
    lse_n = mx_n + jnp.log(jnp.sum(jnp.exp(ln - mx_n), axis=1, keepdims=True))
    iota_n = lax.broadcasted_iota(jnp.int32, (B, NUM_CLASSES), 1)
    tl_n = jnp.sum(jnp.where(iota_n == nat_ref[...], ln, 0.0),
                   axis=1, keepdims=True)
    num_loss = jnp.mean(lse_n - tl_n)

    la = jnp.dot(z2, watom_ref[...], preferred_element_type=f32) + b_atom
    mx_a = jnp.max(la, axis=1, keepdims=True)
    lse_a = mx_a + jnp.log(jnp.sum(jnp.exp(la - mx_a), axis=1, keepdims=True))
    inv_n = 1.0 / nat_ref[...].astype(f32)
    g_ref[:, 0:N_ATOM_CLASSES] = la * (inv_n * (1.0 / B))

    total = cos_loss + latt_loss + kld + num_loss + jnp.mean(lse_a)
    partial_ref[...] = total.reshape(1, 1)


def _make_sc_kernel(n_atoms, c_per_w):
    mesh = plsc.VectorSubcoreMesh(core_axis_name="c", subcore_axis_name="s")

    @functools.partial(
        pl.kernel,
        out_type=jax.ShapeDtypeStruct((_NW, _L), jnp.float32),
        mesh=mesh,
        scratch_types=[
            pltpu.VMEM((c_per_w,), jnp.int32),
            pltpu.VMEM((c_per_w,), jnp.int32),
            pltpu.VMEM((c_per_w,), jnp.int32),
            pltpu.VMEM((c_per_w,), jnp.float32),
            pltpu.VMEM((_L,), jnp.float32),
            pltpu.SemaphoreType.DMA,
            pltpu.SemaphoreType.DMA,
        ],
    )
    def sc_gather_sum(g_hbm, b_hbm, a_hbm, out_hbm, bv, av, fv, vv, accv,
                      sem_in, sem_g):
        wid = lax.axis_index("s") * _NC + lax.axis_index("c")
        base = wid * c_per_w
        cb = pltpu.async_copy(b_hbm.at[pl.ds(base, c_per_w)], bv, sem_in)
        ca = pltpu.async_copy(a_hbm.at[pl.ds(base, c_per_w)], av, sem_in)
        cb.wait()
        ca.wait()

        def fire(j, carry):
            for k in range(128 // _L):
                s = pl.ds(j * 128 + k * _L, _L)
                fv[s] = bv[s] * GL + av[s] - 1
            s128 = pl.ds(j * 128, 128)
            pltpu.async_copy(g_hbm.at[fv.at[s128]], vv.at[s128], sem_g)
            return carry

        lax.fori_loop(0, c_per_w // 128, fire, 0)

        pltpu.make_async_copy(g_hbm.at[pl.ds(0, c_per_w)], vv, sem_g).wait()

        nvalid = n_atoms - base

        def abody(j, acc):
            for k in range(128 // _L):
                off = j * 128 + k * _L
                lane = lax.iota(jnp.int32, _L) + off
                acc = acc + jnp.where(lane < nvalid, vv[pl.ds(off, _L)], 0.0)
            return acc

        acc = lax.fori_loop(0, c_per_w // 128, abody,
                            jnp.zeros((_L,), jnp.float32))
        accv[...] = acc
        pltpu.sync_copy(accv, out_hbm.at[wid])

    return sc_gather_sum


def kernel(z1, z2_raw, eps, num_atoms, atomic_nums, batch, lscaled_lattice,
           W_mu, b_mu, W_sigma, b_sigma, W_latt, b_latt, W_atom, b_atom,
           W_num, b_num, W_p1, b_p1, gamma, beta, W_p2, b_p2,
           scaler_mean, scaler_std):
    f32 = jnp.float32
    n_atoms = atomic_nums.shape[0]
    n_pad = -n_atoms % (_NW * 128)
    c_per_w = (n_atoms + n_pad) // _NW

    def row(v):
        return jnp.pad(v.astype(f32), (0, D - v.shape[0]))[None, :]

    packed = jnp.concatenate([
        row(b_mu), row(b_sigma), row(b_p1), row(gamma), row(beta), row(b_p2),
        row(b_latt), row(b_atom), row(b_num),
        row(scaler_mean), row(scaler_std),
    ], axis=0)

    partial, g = pl.pallas_call(
        _tc_body,
        out_shape=[
            jax.ShapeDtypeStruct((1, 1), f32),
            jax.ShapeDtypeStruct((B, GL), f32),
        ],
    )(z1, z2_raw, eps,
      num_atoms.astype(jnp.int32).reshape(B, 1),
      lscaled_lattice,
      W_mu, W_sigma, W_latt, W_atom, W_num, W_p1, W_p2, packed)

    batch_p = jnp.concatenate([batch, jnp.zeros((n_pad,), jnp.int32)])
    anum_p = jnp.concatenate([atomic_nums, jnp.ones((n_pad,), jnp.int32)])

    sc_parts = _make_sc_kernel(n_atoms, c_per_w)(
        g.reshape(B * GL), batch_p, anum_p)

    return partial[0, 0] - jnp.sum(sc_parts)
